# baseline (speedup 1.0000x reference)
import functools

import jax
import jax.numpy as jnp
from jax import lax
from jax.experimental import pallas as pl
from jax.experimental.pallas import tpu as pltpu
from jax.experimental.pallas import tpu_sc as plsc

D_MODEL = 1024
NUM_TYPES = 8
B_TOTAL = 4 * 8192

NUM_CORES = 2
NUM_SUBCORES = 16
NUM_WORKERS = NUM_CORES * NUM_SUBCORES
B_PER_W = B_TOTAL // NUM_WORKERS
LANES = 16
CHUNK = 16
N_CHUNKS = B_PER_W // CHUNK
N_PAIRS = N_CHUNKS // 2
COL_STEPS = D_MODEL // LANES


@functools.partial(
    pl.kernel,
    mesh=plsc.VectorSubcoreMesh(core_axis_name="c", subcore_axis_name="s"),
    out_type=jax.ShapeDtypeStruct((B_TOTAL, D_MODEL), jnp.float32),
    scratch_types=[
        pltpu.VMEM((B_PER_W,), jnp.int32),
        pltpu.VMEM((NUM_TYPES, D_MODEL), jnp.float32),
        pltpu.VMEM((CHUNK, D_MODEL), jnp.float32),
        pltpu.VMEM((CHUNK, D_MODEL), jnp.float32),
        pltpu.VMEM((CHUNK, D_MODEL), jnp.float32),
        pltpu.VMEM((CHUNK, D_MODEL), jnp.float32),
        pltpu.SemaphoreType.DMA,
        pltpu.SemaphoreType.DMA,
        pltpu.SemaphoreType.DMA,
        pltpu.SemaphoreType.DMA,
    ],
)
def _emb_lookup(
    idx_hbm, table_hbm, out_hbm, idx_v, table_v, buf0, buf1, buf2, buf3,
    s0, s1, s2, s3
):
    wid = lax.axis_index("s") * NUM_CORES + lax.axis_index("c")
    base = wid * B_PER_W
    pltpu.sync_copy(idx_hbm.at[pl.ds(base, B_PER_W)], idx_v)
    pltpu.sync_copy(table_hbm, table_v)

    bufs = (buf0, buf1, buf2, buf3)
    ssems = (s0, s1, s2, s3)

    def fill_chunk(i, b):
        buf = bufs[b]
        vec = idx_v[pl.ds(i * CHUNK, CHUNK)]
        rows = [table_v.at[vec[j]] for j in range(CHUNK)]

        @plsc.parallel_loop(0, COL_STEPS, unroll=16)
        def col_body(u):
            c = u * LANES
            for j in range(CHUNK):
                buf[j, pl.ds(c, LANES)] = rows[j][pl.ds(c, LANES)]

    def start_store(i, b):
        pltpu.async_copy(
            bufs[b], out_hbm.at[pl.ds(base + i * CHUNK, CHUNK)], ssems[b]
        )

    def wait_store(b):
        pltpu.make_async_copy(
            bufs[b], out_hbm.at[pl.ds(base, CHUNK)], ssems[b]
        ).wait()

    def body(q, carry):
        for b in range(4):
            i = 4 * q + b

            @pl.when(q >= 1)
            def _():
                wait_store(b)

            fill_chunk(i, b)
            start_store(i, b)
        return carry

    lax.fori_loop(0, N_CHUNKS // 4, body, 0)

    for b in range(4):
        wait_store(b)


def kernel(token_types, type_embedding_weight):
    flat_idx = token_types.reshape(B_TOTAL).astype(jnp.int32)
    out = _emb_lookup(flat_idx, type_embedding_weight)
    return out.reshape(token_types.shape + (D_MODEL,))


# speedup vs baseline: 1.1753x; 1.1753x over previous
import functools

import jax
import jax.numpy as jnp
from jax import lax
from jax.experimental import pallas as pl
from jax.experimental.pallas import tpu as pltpu
from jax.experimental.pallas import tpu_sc as plsc

D_MODEL = 1024
NUM_TYPES = 8
B_TOTAL = 4 * 8192

NUM_CORES = 2
NUM_SUBCORES = 16
NUM_WORKERS = NUM_CORES * NUM_SUBCORES
B_PER_W = B_TOTAL // NUM_WORKERS
LANES = 16
CHUNK = 16
N_CHUNKS = B_PER_W // CHUNK
N_PAIRS = N_CHUNKS // 2
COL_STEPS = D_MODEL // LANES


@functools.partial(
    pl.kernel,
    mesh=plsc.VectorSubcoreMesh(core_axis_name="c", subcore_axis_name="s"),
    out_type=jax.ShapeDtypeStruct((B_TOTAL, D_MODEL), jnp.float32),
    scratch_types=[
        pltpu.VMEM((B_PER_W,), jnp.int32),
        pltpu.VMEM((NUM_TYPES, D_MODEL), jnp.float32),
        pltpu.VMEM((CHUNK, D_MODEL), jnp.float32),
        pltpu.VMEM((CHUNK, D_MODEL), jnp.float32),
        pltpu.SemaphoreType.DMA,
        pltpu.SemaphoreType.DMA,
    ],
)
def _emb_lookup(idx_hbm, table_hbm, out_hbm, idx_v, table_v, buf0, buf1, s0, s1):
    wid = lax.axis_index("s") * NUM_CORES + lax.axis_index("c")
    base = wid * B_PER_W
    pltpu.sync_copy(idx_hbm.at[pl.ds(base, B_PER_W)], idx_v)
    pltpu.sync_copy(table_hbm, table_v)

    bufs = (buf0, buf1)
    ssems = (s0, s1)

    def fill_chunk(i, b):
        buf = bufs[b]
        vec = idx_v[pl.ds(i * CHUNK, CHUNK)]
        rows = [table_v.at[vec[j]] for j in range(CHUNK)]

        @plsc.parallel_loop(0, COL_STEPS, unroll=8)
        def col_body(u):
            c = u * LANES
            for j in range(CHUNK):
                buf[j, pl.ds(c, LANES)] = rows[j][pl.ds(c, LANES)]

    def start_store(i, b):
        pltpu.async_copy(
            bufs[b], out_hbm.at[pl.ds(base + i * CHUNK, CHUNK)], ssems[b]
        )

    def wait_store(b):
        pltpu.make_async_copy(
            bufs[b], out_hbm.at[pl.ds(base, CHUNK)], ssems[b]
        ).wait()

    def body(p, carry):
        for b in range(2):
            i = 2 * p + b

            @pl.when(p >= 1)
            def _():
                wait_store(b)

            fill_chunk(i, b)
            start_store(i, b)
        return carry

    lax.fori_loop(0, N_PAIRS, body, 0)

    for b in range(2):
        wait_store(b)


def kernel(token_types, type_embedding_weight):
    flat_idx = token_types.reshape(B_TOTAL).astype(jnp.int32)
    out = _emb_lookup(flat_idx, type_embedding_weight)
    return out.reshape(token_types.shape + (D_MODEL,))
